# baseline (speedup 1.0000x reference)
import functools

import jax
import jax.numpy as jnp
from jax import lax
from jax.experimental import pallas as pl
from jax.experimental.pallas import tpu as pltpu
from jax.experimental.pallas import tpu_sc as plsc

N = 10000
E = 320000
D_IN = 128
D_H = 128
D_Z = 32

NC = 2
NS = 16
NW = NC * NS
EB = 128
NB_W = 80
NBLK = E // EB
NB_LAST = NBLK - (NW - 1) * NB_W
NP = 10240
RPT = 640

BR = 2000


def _worker_id():
    return lax.axis_index("s") * NC + lax.axis_index("c")


def _deg_sc(dstb, deg_out, didx, ones_v, tmpd, acc, sem):
    c = lax.axis_index("c")
    s = lax.axis_index("s")
    w = _worker_id()
    off = pl.multiple_of(s * RPT, 8)

    def zfill(i, carry):
        tmpd[pl.ds(i * 16, 16)] = jnp.zeros((16,), jnp.float32)
        return carry
    lax.fori_loop(0, RPT // 16, zfill, 0)
    for j in range(EB // 16):
        ones_v[pl.ds(j * 16, 16)] = jnp.ones((16,), jnp.float32)
    pltpu.sync_copy(tmpd, acc.at[pl.ds(off, RPT)])
    plsc.subcore_barrier()

    nb = jnp.where(w == NW - 1, NB_LAST, NB_W)

    @pl.when(w < NW - 1)
    def _():
        pltpu.sync_copy(dstb.at[pl.ds(w * NB_W, NB_W)], didx)

    @pl.when(w == NW - 1)
    def _():
        pltpu.sync_copy(dstb.at[pl.ds((NW - 1) * NB_W, NB_LAST)],
                        didx.at[pl.ds(0, NB_LAST)])

    def fire(i, carry):
        pltpu.async_copy(ones_v, acc.at[didx.at[i]], sem, add=True)
        return carry
    lax.fori_loop(0, nb, fire, 0)

    def drain(i, carry):
        pltpu.make_async_copy(ones_v, acc.at[didx.at[0]], sem).wait()
        return carry
    lax.fori_loop(0, nb, drain, 0)
    plsc.subcore_barrier()

    pltpu.sync_copy(acc.at[pl.ds(off, RPT)], tmpd)
    pltpu.sync_copy(tmpd, deg_out.at[c, pl.ds(off, RPT)])


def _deg_call(dstb):
    kfn = pl.kernel(
        _deg_sc,
        out_type=jax.ShapeDtypeStruct((NC, NP), jnp.float32),
        mesh=plsc.VectorSubcoreMesh(
            core_axis_name="c", subcore_axis_name="s",
            num_cores=NC, num_subcores=NS),
        scratch_types=[
            pltpu.VMEM((NB_W, EB), jnp.int32),
            pltpu.VMEM((EB,), jnp.float32),
            pltpu.VMEM((RPT,), jnp.float32),
            pltpu.VMEM_SHARED((NP,), jnp.float32),
            pltpu.SemaphoreType.DMA,
        ],
        compiler_params=pltpu.CompilerParams(use_tc_tiling_on_sc=False),
    )
    return kfn(dstb)


NBUF = 4


def _prop_sc(D, srcb, dstb, tbl, out, sidx, didx, rows, tmp,
             gsems, ssems, acc):
    c = lax.axis_index("c")
    s = lax.axis_index("s")
    w = _worker_id()
    off = pl.multiple_of(s * RPT, 8)

    def zrow(r, carry):
        for j in range(D // 16):
            rows[0][r, pl.ds(j * 16, 16)] = jnp.zeros((16,), jnp.float32)
        return carry
    lax.fori_loop(0, EB, zrow, 0)
    for q in range(RPT // EB):
        pltpu.sync_copy(rows[0], acc.at[pl.ds(off + q * EB, EB)])
    plsc.subcore_barrier()

    nb = jnp.where(w == NW - 1, NB_LAST, NB_W)

    @pl.when(w < NW - 1)
    def _():
        pltpu.sync_copy(srcb.at[pl.ds(w * NB_W, NB_W)], sidx)
        pltpu.sync_copy(dstb.at[pl.ds(w * NB_W, NB_W)], didx)

    @pl.when(w == NW - 1)
    def _():
        pltpu.sync_copy(srcb.at[pl.ds((NW - 1) * NB_W, NB_LAST)],
                        sidx.at[pl.ds(0, NB_LAST)])
        pltpu.sync_copy(dstb.at[pl.ds((NW - 1) * NB_W, NB_LAST)],
                        didx.at[pl.ds(0, NB_LAST)])

    def g_start(i, j):
        pltpu.async_copy(tbl.at[sidx.at[i]], rows[j], gsems[j])

    def g_wait(i, j):
        pltpu.make_async_copy(tbl.at[sidx.at[i]], rows[j], gsems[j]).wait()

    def s_start(i, j):
        pltpu.async_copy(rows[j], acc.at[didx.at[i]], ssems[j], add=True)

    def s_wait(i, j):
        pltpu.make_async_copy(rows[j], acc.at[didx.at[i]], ssems[j]).wait()

    for j in range(NBUF):
        g_start(j, j)

    def body(k, carry):
        base = NBUF * k
        for j in range(NBUF):
            g_wait(base + j, j)
            s_start(base + j, j)
        for j in range(NBUF):
            s_wait(base + j, j)

            @pl.when(base + NBUF + j < nb)
            def _():
                g_start(base + NBUF + j, j)
        return carry
    lax.fori_loop(0, nb // NBUF, body, 0)
    plsc.subcore_barrier()

    CH = 320
    for q in range(RPT // CH):
        pltpu.sync_copy(acc.at[pl.ds(off + q * CH, CH)], tmp)
        pltpu.sync_copy(tmp, out.at[c, pl.ds(off + q * CH, CH)])


def _prop_call(srcb, dstb, tbl, D):
    kfn = pl.kernel(
        functools.partial(_prop_sc, D),
        out_type=jax.ShapeDtypeStruct((NC, NP, D), jnp.float32),
        mesh=plsc.VectorSubcoreMesh(
            core_axis_name="c", subcore_axis_name="s",
            num_cores=NC, num_subcores=NS),
        scratch_types=[
            pltpu.VMEM((NB_W, EB), jnp.int32),
            pltpu.VMEM((NB_W, EB), jnp.int32),
            [pltpu.VMEM((EB, D), jnp.float32) for _ in range(NBUF)],
            pltpu.VMEM((320, D), jnp.float32),
            [pltpu.SemaphoreType.DMA for _ in range(NBUF)],
            [pltpu.SemaphoreType.DMA for _ in range(NBUF)],
            pltpu.VMEM_SHARED((NP, D), jnp.float32),
        ],
        compiler_params=pltpu.CompilerParams(use_tc_tiling_on_sc=False),
    )
    return kfn(srcb, dstb, tbl)


DQ = D_H // 2


def _enc1_tc(deg_ref, x_ref, w1_ref, b1_ref, t1a_ref, t1b_ref, dinv_ref,
             lib_ref):
    deg = deg_ref[0] + deg_ref[1] + 1.0
    dinv = lax.rsqrt(deg)
    x = x_ref[...]
    p = jnp.dot(x, w1_ref[...], preferred_element_type=jnp.float32) + b1_ref[...]
    t1 = p * dinv
    t1a_ref[...] = t1[:, :DQ]
    t1b_ref[...] = t1[:, DQ:]
    dinv_ref[...] = dinv
    lib_ref[...] = jnp.sum(x, axis=1, keepdims=True)


def _enc1_call(deg, x, W1, b1):
    grid = N // BR
    return pl.pallas_call(
        _enc1_tc,
        grid=(grid,),
        in_specs=[
            pl.BlockSpec((NC, BR, 1), lambda i: (0, i, 0)),
            pl.BlockSpec((BR, D_IN), lambda i: (i, 0)),
            pl.BlockSpec((D_IN, D_H), lambda i: (0, 0)),
            pl.BlockSpec((1, D_H), lambda i: (0, 0)),
        ],
        out_specs=[
            pl.BlockSpec((BR, DQ), lambda i: (i, 0)),
            pl.BlockSpec((BR, DQ), lambda i: (i, 0)),
            pl.BlockSpec((BR, 1), lambda i: (i, 0)),
            pl.BlockSpec((BR, 1), lambda i: (i, 0)),
        ],
        out_shape=[
            jax.ShapeDtypeStruct((N, DQ), jnp.float32),
            jax.ShapeDtypeStruct((N, DQ), jnp.float32),
            jax.ShapeDtypeStruct((N, 1), jnp.float32),
            jax.ShapeDtypeStruct((N, 1), jnp.float32),
        ],
    )(deg, x, W1, b1)


def _enc2_tc(s1a_ref, s1b_ref, t1a_ref, t1b_ref, dinv_ref, wc_ref, bc_ref,
             t2_ref):
    dinv = dinv_ref[...]
    ha = (s1a_ref[0] + s1a_ref[1] + t1a_ref[...]) * dinv
    hb = (s1b_ref[0] + s1b_ref[1] + t1b_ref[...]) * dinv
    h = jnp.maximum(jnp.concatenate([ha, hb], axis=1), 0.0)
    p = jnp.dot(h, wc_ref[...], preferred_element_type=jnp.float32) + bc_ref[...]
    t2_ref[...] = p * dinv


def _enc2_call(s1a, s1b, t1a, t1b, dinv, Wc, bc):
    grid = N // BR
    return pl.pallas_call(
        _enc2_tc,
        grid=(grid,),
        in_specs=[
            pl.BlockSpec((NC, BR, DQ), lambda i: (0, i, 0)),
            pl.BlockSpec((NC, BR, DQ), lambda i: (0, i, 0)),
            pl.BlockSpec((BR, DQ), lambda i: (i, 0)),
            pl.BlockSpec((BR, DQ), lambda i: (i, 0)),
            pl.BlockSpec((BR, 1), lambda i: (i, 0)),
            pl.BlockSpec((D_H, 2 * D_Z), lambda i: (0, 0)),
            pl.BlockSpec((1, 2 * D_Z), lambda i: (0, 0)),
        ],
        out_specs=pl.BlockSpec((BR, 2 * D_Z), lambda i: (i, 0)),
        out_shape=jax.ShapeDtypeStruct((N, 2 * D_Z), jnp.float32),
    )(s1a, s1b, t1a, t1b, dinv, Wc, bc)


def _dec_tc(s2_ref, t2_ref, dinv_ref, eps_ref, lib_ref, wnb_ref, wzi_ref,
            mask_ref, mu_ref, logstd_ref, z_ref, nb_ref, zi_ref):
    g = (s2_ref[0] + s2_ref[1] + t2_ref[...]) * dinv_ref[...]
    mu = g[:, :D_Z]
    logstd = g[:, D_Z:]
    z = mu + eps_ref[...] * jnp.exp(logstd)
    mu_ref[...] = mu
    logstd_ref[...] = logstd
    z_ref[...] = z
    wnb = wnb_ref[...] * mask_ref[...]
    wzi = wzi_ref[...] * mask_ref[...]
    nb_logits = jnp.dot(z, wnb, preferred_element_type=jnp.float32)
    m = jnp.max(nb_logits, axis=1, keepdims=True)
    ex = jnp.exp(nb_logits - m)
    nb_ref[...] = lib_ref[...] * ex / jnp.sum(ex, axis=1, keepdims=True)
    zi_ref[...] = jnp.dot(z, wzi, preferred_element_type=jnp.float32)


def _dec_call(s2, t2, dinv, eps, lib, W_nb, W_zi, mask):
    grid = N // BR
    return pl.pallas_call(
        _dec_tc,
        grid=(grid,),
        in_specs=[
            pl.BlockSpec((NC, BR, 2 * D_Z), lambda i: (0, i, 0)),
            pl.BlockSpec((BR, 2 * D_Z), lambda i: (i, 0)),
            pl.BlockSpec((BR, 1), lambda i: (i, 0)),
            pl.BlockSpec((BR, D_Z), lambda i: (i, 0)),
            pl.BlockSpec((BR, 1), lambda i: (i, 0)),
            pl.BlockSpec((D_Z, D_IN), lambda i: (0, 0)),
            pl.BlockSpec((D_Z, D_IN), lambda i: (0, 0)),
            pl.BlockSpec((D_Z, D_IN), lambda i: (0, 0)),
        ],
        out_specs=[
            pl.BlockSpec((BR, D_Z), lambda i: (i, 0)),
            pl.BlockSpec((BR, D_Z), lambda i: (i, 0)),
            pl.BlockSpec((BR, D_Z), lambda i: (i, 0)),
            pl.BlockSpec((BR, D_IN), lambda i: (i, 0)),
            pl.BlockSpec((BR, D_IN), lambda i: (i, 0)),
        ],
        out_shape=[
            jax.ShapeDtypeStruct((N, D_Z), jnp.float32),
            jax.ShapeDtypeStruct((N, D_Z), jnp.float32),
            jax.ShapeDtypeStruct((N, D_Z), jnp.float32),
            jax.ShapeDtypeStruct((N, D_IN), jnp.float32),
            jax.ShapeDtypeStruct((N, D_IN), jnp.float32),
        ],
    )(s2, t2, dinv, eps, lib, W_nb, W_zi, mask)


def _adj_tc(zr_ref, zc_ref, adj_ref):
    adj_ref[...] = lax.dot_general(
        zr_ref[...], zc_ref[...], (((1,), (1,)), ((), ())),
        preferred_element_type=jnp.float32)


def _adj_call(z):
    BRR = 400
    return pl.pallas_call(
        _adj_tc,
        grid=(N // BRR,),
        in_specs=[
            pl.BlockSpec((BRR, D_Z), lambda i: (i, 0)),
            pl.BlockSpec((N, D_Z), lambda i: (0, 0)),
        ],
        out_specs=pl.BlockSpec((BRR, N), lambda i: (i, 0)),
        out_shape=jax.ShapeDtypeStruct((N, N), jnp.float32),
    )(z, z)


def kernel(x, edge_index, W1, b1, W_mu, b_mu, W_logstd, b_logstd,
           W_nb, W_zi, mask, eps):
    srcb = edge_index[0].reshape(NBLK, EB)
    dstb = edge_index[1].reshape(NBLK, EB)

    deg = _deg_call(dstb).reshape(NC, NP, 1)
    t1a, t1b, dinv, lib = _enc1_call(deg, x, W1, b1.reshape(1, D_H))
    s1a = _prop_call(srcb, dstb, t1a, DQ)
    s1b = _prop_call(srcb, dstb, t1b, DQ)
    Wc = jnp.concatenate([W_mu, W_logstd], axis=1)
    bc = jnp.concatenate([b_mu, b_logstd]).reshape(1, 2 * D_Z)
    t2 = _enc2_call(s1a, s1b, t1a, t1b, dinv, Wc, bc)
    s2 = _prop_call(srcb, dstb, t2, 2 * D_Z)
    mu, logstd, z, nb_means, zi = _dec_call(
        s2, t2, dinv, eps, lib, W_nb, W_zi, mask)
    adj = _adj_call(z)
    return (adj, nb_means, zi, mu, logstd)


# speedup vs baseline: 1.0750x; 1.0750x over previous
import functools

import jax
import jax.numpy as jnp
from jax import lax
from jax.experimental import pallas as pl
from jax.experimental.pallas import tpu as pltpu
from jax.experimental.pallas import tpu_sc as plsc

N = 10000
E = 320000
D_IN = 128
D_H = 128
D_Z = 32

NC = 2
NS = 16
NW = NC * NS
EB = 128
NB_W = 80
NBLK = E // EB
NB_LAST = NBLK - (NW - 1) * NB_W
NP = 10240
RPT = 640

BR = 2000


def _worker_id():
    return lax.axis_index("s") * NC + lax.axis_index("c")


def _deg_sc(dstb, deg_out, didx, ones_v, tmpd, acc, sem):
    c = lax.axis_index("c")
    s = lax.axis_index("s")
    w = _worker_id()
    off = pl.multiple_of(s * RPT, 8)

    def zfill(i, carry):
        tmpd[pl.ds(i * 16, 16)] = jnp.zeros((16,), jnp.float32)
        return carry
    lax.fori_loop(0, RPT // 16, zfill, 0)
    for j in range(EB // 16):
        ones_v[pl.ds(j * 16, 16)] = jnp.ones((16,), jnp.float32)
    pltpu.sync_copy(tmpd, acc.at[pl.ds(off, RPT)])
    plsc.subcore_barrier()

    nb = jnp.where(w == NW - 1, NB_LAST, NB_W)

    @pl.when(w < NW - 1)
    def _():
        pltpu.sync_copy(dstb.at[pl.ds(w * NB_W, NB_W)], didx)

    @pl.when(w == NW - 1)
    def _():
        pltpu.sync_copy(dstb.at[pl.ds((NW - 1) * NB_W, NB_LAST)],
                        didx.at[pl.ds(0, NB_LAST)])

    def fire(i, carry):
        pltpu.async_copy(ones_v, acc.at[didx.at[i]], sem, add=True)
        return carry
    lax.fori_loop(0, nb, fire, 0)

    def drain(i, carry):
        pltpu.make_async_copy(ones_v, acc.at[didx.at[0]], sem).wait()
        return carry
    lax.fori_loop(0, nb, drain, 0)
    plsc.subcore_barrier()

    pltpu.sync_copy(acc.at[pl.ds(off, RPT)], tmpd)
    pltpu.sync_copy(tmpd, deg_out.at[c, pl.ds(off, RPT)])


def _deg_call(dstb):
    kfn = pl.kernel(
        _deg_sc,
        out_type=jax.ShapeDtypeStruct((NC, NP), jnp.float32),
        mesh=plsc.VectorSubcoreMesh(
            core_axis_name="c", subcore_axis_name="s",
            num_cores=NC, num_subcores=NS),
        scratch_types=[
            pltpu.VMEM((NB_W, EB), jnp.int32),
            pltpu.VMEM((EB,), jnp.float32),
            pltpu.VMEM((RPT,), jnp.float32),
            pltpu.VMEM_SHARED((NP,), jnp.float32),
            pltpu.SemaphoreType.DMA,
        ],
        compiler_params=pltpu.CompilerParams(use_tc_tiling_on_sc=False),
    )
    return kfn(dstb)


NBUF = 4


def _prop_sc(D, srcb, dstb, tbl, out, sidx, didx, rows, tmp,
             gsems, ssems, acc):
    c = lax.axis_index("c")
    s = lax.axis_index("s")
    w = _worker_id()
    off = pl.multiple_of(s * RPT, 8)

    def zrow(r, carry):
        for j in range(D // 16):
            rows[0][r, pl.ds(j * 16, 16)] = jnp.zeros((16,), jnp.float32)
        return carry
    lax.fori_loop(0, EB, zrow, 0)
    for q in range(RPT // EB):
        pltpu.sync_copy(rows[0], acc.at[pl.ds(off + q * EB, EB)])
    plsc.subcore_barrier()

    nb = jnp.where(w == NW - 1, NB_LAST, NB_W)

    @pl.when(w < NW - 1)
    def _():
        pltpu.sync_copy(srcb.at[pl.ds(w * NB_W, NB_W)], sidx)
        pltpu.sync_copy(dstb.at[pl.ds(w * NB_W, NB_W)], didx)

    @pl.when(w == NW - 1)
    def _():
        pltpu.sync_copy(srcb.at[pl.ds((NW - 1) * NB_W, NB_LAST)],
                        sidx.at[pl.ds(0, NB_LAST)])
        pltpu.sync_copy(dstb.at[pl.ds((NW - 1) * NB_W, NB_LAST)],
                        didx.at[pl.ds(0, NB_LAST)])

    def g_start(i, j):
        pltpu.async_copy(tbl.at[sidx.at[i]], rows[j], gsems[j])

    def g_wait(i, j):
        pltpu.make_async_copy(tbl.at[sidx.at[i]], rows[j], gsems[j]).wait()

    def s_start(i, j):
        pltpu.async_copy(rows[j], acc.at[didx.at[i]], ssems[j], add=True)

    def s_wait(i, j):
        pltpu.make_async_copy(rows[j], acc.at[didx.at[i]], ssems[j]).wait()

    for j in range(NBUF):
        g_start(j, j)

    def body(k, carry):
        base = NBUF * k
        for j in range(NBUF):
            g_wait(base + j, j)
            s_start(base + j, j)
        for j in range(NBUF):
            s_wait(base + j, j)

            @pl.when(base + NBUF + j < nb)
            def _():
                g_start(base + NBUF + j, j)
        return carry
    lax.fori_loop(0, nb // NBUF, body, 0)
    plsc.subcore_barrier()

    CH = 320
    for q in range(RPT // CH):
        pltpu.sync_copy(acc.at[pl.ds(off + q * CH, CH)], tmp)
        pltpu.sync_copy(tmp, out.at[c, pl.ds(off + q * CH, CH), pl.ds(0, D)])


def _prop_call(srcb, dstb, tbl, D):
    kfn = pl.kernel(
        functools.partial(_prop_sc, D),
        out_type=jax.ShapeDtypeStruct((NC, NP, EB), jnp.float32),
        mesh=plsc.VectorSubcoreMesh(
            core_axis_name="c", subcore_axis_name="s",
            num_cores=NC, num_subcores=NS),
        scratch_types=[
            pltpu.VMEM((NB_W, EB), jnp.int32),
            pltpu.VMEM((NB_W, EB), jnp.int32),
            [pltpu.VMEM((EB, D), jnp.float32) for _ in range(NBUF)],
            pltpu.VMEM((320, D), jnp.float32),
            [pltpu.SemaphoreType.DMA for _ in range(NBUF)],
            [pltpu.SemaphoreType.DMA for _ in range(NBUF)],
            pltpu.VMEM_SHARED((NP, D), jnp.float32),
        ],
        compiler_params=pltpu.CompilerParams(use_tc_tiling_on_sc=False),
    )
    return kfn(srcb, dstb, tbl)


DQ = D_H // 2


def _enc1_tc(dinv_ref, x_ref, w1_ref, b1_ref, t1a_ref, t1b_ref, lib_ref):
    dinv = dinv_ref[...]
    x = x_ref[...]
    p = jnp.dot(x, w1_ref[...], preferred_element_type=jnp.float32) + b1_ref[...]
    t1 = p * dinv
    t1a_ref[...] = t1[:, :DQ]
    t1b_ref[...] = t1[:, DQ:]
    lib_ref[...] = jnp.sum(x, axis=1, keepdims=True)


def _enc1_call(dinv, x, W1, b1):
    grid = N // BR
    return pl.pallas_call(
        _enc1_tc,
        grid=(grid,),
        in_specs=[
            pl.BlockSpec((BR, 1), lambda i: (i, 0)),
            pl.BlockSpec((BR, D_IN), lambda i: (i, 0)),
            pl.BlockSpec((D_IN, D_H), lambda i: (0, 0)),
            pl.BlockSpec((1, D_H), lambda i: (0, 0)),
        ],
        out_specs=[
            pl.BlockSpec((BR, DQ), lambda i: (i, 0)),
            pl.BlockSpec((BR, DQ), lambda i: (i, 0)),
            pl.BlockSpec((BR, 1), lambda i: (i, 0)),
        ],
        out_shape=[
            jax.ShapeDtypeStruct((N, DQ), jnp.float32),
            jax.ShapeDtypeStruct((N, DQ), jnp.float32),
            jax.ShapeDtypeStruct((N, 1), jnp.float32),
        ],
    )(dinv, x, W1, b1)


def _enc2_tc(s1a_ref, s1b_ref, t1a_ref, t1b_ref, dinv_ref, wc_ref, bc_ref,
             t2_ref):
    dinv = dinv_ref[...]
    sa = s1a_ref[0][:, :DQ] + s1a_ref[1][:, :DQ]
    sb = s1b_ref[0][:, :DQ] + s1b_ref[1][:, :DQ]
    ha = (sa + t1a_ref[...]) * dinv
    hb = (sb + t1b_ref[...]) * dinv
    h = jnp.maximum(jnp.concatenate([ha, hb], axis=1), 0.0)
    p = jnp.dot(h, wc_ref[...], preferred_element_type=jnp.float32) + bc_ref[...]
    t2_ref[...] = p * dinv


def _enc2_call(s1a, s1b, t1a, t1b, dinv, Wc, bc):
    grid = N // BR
    return pl.pallas_call(
        _enc2_tc,
        grid=(grid,),
        in_specs=[
            pl.BlockSpec((NC, BR, EB), lambda i: (0, i, 0)),
            pl.BlockSpec((NC, BR, EB), lambda i: (0, i, 0)),
            pl.BlockSpec((BR, DQ), lambda i: (i, 0)),
            pl.BlockSpec((BR, DQ), lambda i: (i, 0)),
            pl.BlockSpec((BR, 1), lambda i: (i, 0)),
            pl.BlockSpec((D_H, 2 * D_Z), lambda i: (0, 0)),
            pl.BlockSpec((1, 2 * D_Z), lambda i: (0, 0)),
        ],
        out_specs=pl.BlockSpec((BR, 2 * D_Z), lambda i: (i, 0)),
        out_shape=jax.ShapeDtypeStruct((N, 2 * D_Z), jnp.float32),
    )(s1a, s1b, t1a, t1b, dinv, Wc, bc)


def _dec_tc(s2_ref, t2_ref, dinv_ref, eps_ref, lib_ref, wnb_ref, wzi_ref,
            mask_ref, mu_ref, logstd_ref, z_ref, nb_ref, zi_ref):
    s2 = s2_ref[0][:, :2 * D_Z] + s2_ref[1][:, :2 * D_Z]
    g = (s2 + t2_ref[...]) * dinv_ref[...]
    mu = g[:, :D_Z]
    logstd = g[:, D_Z:]
    z = mu + eps_ref[...] * jnp.exp(logstd)
    mu_ref[...] = mu
    logstd_ref[...] = logstd
    z_ref[...] = z
    wnb = wnb_ref[...] * mask_ref[...]
    wzi = wzi_ref[...] * mask_ref[...]
    nb_logits = jnp.dot(z, wnb, preferred_element_type=jnp.float32)
    m = jnp.max(nb_logits, axis=1, keepdims=True)
    ex = jnp.exp(nb_logits - m)
    nb_ref[...] = lib_ref[...] * ex / jnp.sum(ex, axis=1, keepdims=True)
    zi_ref[...] = jnp.dot(z, wzi, preferred_element_type=jnp.float32)


def _dec_call(s2, t2, dinv, eps, lib, W_nb, W_zi, mask):
    grid = N // BR
    return pl.pallas_call(
        _dec_tc,
        grid=(grid,),
        in_specs=[
            pl.BlockSpec((NC, BR, EB), lambda i: (0, i, 0)),
            pl.BlockSpec((BR, 2 * D_Z), lambda i: (i, 0)),
            pl.BlockSpec((BR, 1), lambda i: (i, 0)),
            pl.BlockSpec((BR, D_Z), lambda i: (i, 0)),
            pl.BlockSpec((BR, 1), lambda i: (i, 0)),
            pl.BlockSpec((D_Z, D_IN), lambda i: (0, 0)),
            pl.BlockSpec((D_Z, D_IN), lambda i: (0, 0)),
            pl.BlockSpec((D_Z, D_IN), lambda i: (0, 0)),
        ],
        out_specs=[
            pl.BlockSpec((BR, D_Z), lambda i: (i, 0)),
            pl.BlockSpec((BR, D_Z), lambda i: (i, 0)),
            pl.BlockSpec((BR, D_Z), lambda i: (i, 0)),
            pl.BlockSpec((BR, D_IN), lambda i: (i, 0)),
            pl.BlockSpec((BR, D_IN), lambda i: (i, 0)),
        ],
        out_shape=[
            jax.ShapeDtypeStruct((N, D_Z), jnp.float32),
            jax.ShapeDtypeStruct((N, D_Z), jnp.float32),
            jax.ShapeDtypeStruct((N, D_Z), jnp.float32),
            jax.ShapeDtypeStruct((N, D_IN), jnp.float32),
            jax.ShapeDtypeStruct((N, D_IN), jnp.float32),
        ],
    )(s2, t2, dinv, eps, lib, W_nb, W_zi, mask)


def _adj_tc(zr_ref, zc_ref, adj_ref):
    adj_ref[...] = lax.dot_general(
        zr_ref[...], zc_ref[...], (((1,), (1,)), ((), ())),
        preferred_element_type=jnp.float32)


def _adj_call(z):
    BRR = 400
    return pl.pallas_call(
        _adj_tc,
        grid=(N // BRR,),
        in_specs=[
            pl.BlockSpec((BRR, D_Z), lambda i: (i, 0)),
            pl.BlockSpec((N, D_Z), lambda i: (0, 0)),
        ],
        out_specs=pl.BlockSpec((BRR, N), lambda i: (i, 0)),
        out_shape=jax.ShapeDtypeStruct((N, N), jnp.float32),
    )(z, z)


def kernel(x, edge_index, W1, b1, W_mu, b_mu, W_logstd, b_logstd,
           W_nb, W_zi, mask, eps):
    srcb = edge_index[0].reshape(NBLK, EB)
    dstb = edge_index[1].reshape(NBLK, EB)

    deg = _deg_call(dstb)
    dinv = lax.rsqrt(deg[0, :N] + deg[1, :N] + 1.0).reshape(N, 1)
    t1a, t1b, lib = _enc1_call(dinv, x, W1, b1.reshape(1, D_H))
    s1a = _prop_call(srcb, dstb, t1a, DQ)
    s1b = _prop_call(srcb, dstb, t1b, DQ)
    Wc = jnp.concatenate([W_mu, W_logstd], axis=1)
    bc = jnp.concatenate([b_mu, b_logstd]).reshape(1, 2 * D_Z)
    t2 = _enc2_call(s1a, s1b, t1a, t1b, dinv, Wc, bc)
    s2 = _prop_call(srcb, dstb, t2, 2 * D_Z)
    mu, logstd, z, nb_means, zi = _dec_call(
        s2, t2, dinv, eps, lib, W_nb, W_zi, mask)
    adj = _adj_call(z)
    return (adj, nb_means, zi, mu, logstd)


# speedup vs baseline: 1.1201x; 1.0420x over previous
import functools

import jax
import jax.numpy as jnp
from jax import lax
from jax.experimental import pallas as pl
from jax.experimental.pallas import tpu as pltpu
from jax.experimental.pallas import tpu_sc as plsc

N = 10000
E = 320000
D_IN = 128
D_H = 128
D_Z = 32

NC = 2
NS = 16
NW = NC * NS
EB = 128
NB_W = 80
NBLK = E // EB
NB_LAST = NBLK - (NW - 1) * NB_W
NP = 10240
RPT = 640

BR = 2000


def _worker_id():
    return lax.axis_index("s") * NC + lax.axis_index("c")


def _deg_sc(eib, deg_out, didx, ones_v, tmpd, acc, sem):
    c = lax.axis_index("c")
    s = lax.axis_index("s")
    w = _worker_id()
    off = pl.multiple_of(s * RPT, 8)

    def zfill(i, carry):
        tmpd[pl.ds(i * 16, 16)] = jnp.zeros((16,), jnp.float32)
        return carry
    lax.fori_loop(0, RPT // 16, zfill, 0)
    for j in range(EB // 16):
        ones_v[pl.ds(j * 16, 16)] = jnp.ones((16,), jnp.float32)
    pltpu.sync_copy(tmpd, acc.at[pl.ds(off, RPT)])
    plsc.subcore_barrier()

    nb = jnp.where(w == NW - 1, NB_LAST, NB_W)

    @pl.when(w < NW - 1)
    def _():
        pltpu.sync_copy(eib.at[1, pl.ds(w * NB_W, NB_W)], didx)

    @pl.when(w == NW - 1)
    def _():
        pltpu.sync_copy(eib.at[1, pl.ds((NW - 1) * NB_W, NB_LAST)],
                        didx.at[pl.ds(0, NB_LAST)])

    def fire(i, carry):
        pltpu.async_copy(ones_v, acc.at[didx.at[i]], sem, add=True)
        return carry
    lax.fori_loop(0, nb, fire, 0)

    def drain(i, carry):
        pltpu.make_async_copy(ones_v, acc.at[didx.at[0]], sem).wait()
        return carry
    lax.fori_loop(0, nb, drain, 0)
    plsc.subcore_barrier()

    pltpu.sync_copy(acc.at[pl.ds(off, RPT)], tmpd)
    pltpu.sync_copy(tmpd, deg_out.at[c, pl.ds(off, RPT)])


def _deg_call(eib):
    kfn = pl.kernel(
        _deg_sc,
        out_type=jax.ShapeDtypeStruct((NC, NP), jnp.float32),
        mesh=plsc.VectorSubcoreMesh(
            core_axis_name="c", subcore_axis_name="s",
            num_cores=NC, num_subcores=NS),
        scratch_types=[
            pltpu.VMEM((NB_W, EB), jnp.int32),
            pltpu.VMEM((EB,), jnp.float32),
            pltpu.VMEM((RPT,), jnp.float32),
            pltpu.VMEM_SHARED((NP,), jnp.float32),
            pltpu.SemaphoreType.DMA,
        ],
        compiler_params=pltpu.CompilerParams(use_tc_tiling_on_sc=False),
    )
    return kfn(eib)


NBUF = 4


def _prop_sc(D, MULT, ADD, eib, tbl, out, sidx, didx, rows, tmp,
             gsems, ssems, acc):
    c = lax.axis_index("c")
    s = lax.axis_index("s")
    w = _worker_id()
    off = pl.multiple_of(s * RPT, 8)

    def zrow(r, carry):
        for j in range(D // 16):
            rows[0][r, pl.ds(j * 16, 16)] = jnp.zeros((16,), jnp.float32)
        return carry
    lax.fori_loop(0, EB, zrow, 0)
    for q in range(RPT // EB):
        pltpu.sync_copy(rows[0], acc.at[pl.ds(off + q * EB, EB)])
    plsc.subcore_barrier()

    nb = jnp.where(w == NW - 1, NB_LAST, NB_W)

    @pl.when(w < NW - 1)
    def _():
        pltpu.sync_copy(eib.at[0, pl.ds(w * NB_W, NB_W)], sidx)
        pltpu.sync_copy(eib.at[1, pl.ds(w * NB_W, NB_W)], didx)

    @pl.when(w == NW - 1)
    def _():
        pltpu.sync_copy(eib.at[0, pl.ds((NW - 1) * NB_W, NB_LAST)],
                        sidx.at[pl.ds(0, NB_LAST)])
        pltpu.sync_copy(eib.at[1, pl.ds((NW - 1) * NB_W, NB_LAST)],
                        didx.at[pl.ds(0, NB_LAST)])

    if MULT != 1 or ADD != 0:
        def xform(r, carry):
            for j in range(EB // 16):
                v = sidx[r, pl.ds(j * 16, 16)]
                sidx[r, pl.ds(j * 16, 16)] = v * MULT + ADD
            return carry
        lax.fori_loop(0, NB_W, xform, 0)

    def g_start(i, j):
        pltpu.async_copy(tbl.at[sidx.at[i]], rows[j], gsems[j])

    def g_wait(i, j):
        pltpu.make_async_copy(tbl.at[sidx.at[i]], rows[j], gsems[j]).wait()

    def s_start(i, j):
        pltpu.async_copy(rows[j], acc.at[didx.at[i]], ssems[j], add=True)

    def s_wait(i, j):
        pltpu.make_async_copy(rows[j], acc.at[didx.at[i]], ssems[j]).wait()

    for j in range(NBUF):
        g_start(j, j)

    def body(k, carry):
        base = NBUF * k
        for j in range(NBUF):
            g_wait(base + j, j)
            s_start(base + j, j)
        for j in range(NBUF):
            s_wait(base + j, j)

            @pl.when(base + NBUF + j < nb)
            def _():
                g_start(base + NBUF + j, j)
        return carry
    lax.fori_loop(0, nb // NBUF, body, 0)
    plsc.subcore_barrier()

    CH = 320
    for q in range(RPT // CH):
        pltpu.sync_copy(acc.at[pl.ds(off + q * CH, CH)], tmp)
        pltpu.sync_copy(tmp, out.at[c, pl.ds(off + q * CH, CH), pl.ds(0, D)])


def _prop_call(eib, tbl, D, mult=1, add=0):
    kfn = pl.kernel(
        functools.partial(_prop_sc, D, mult, add),
        out_type=jax.ShapeDtypeStruct((NC, NP, EB), jnp.float32),
        mesh=plsc.VectorSubcoreMesh(
            core_axis_name="c", subcore_axis_name="s",
            num_cores=NC, num_subcores=NS),
        scratch_types=[
            pltpu.VMEM((NB_W, EB), jnp.int32),
            pltpu.VMEM((NB_W, EB), jnp.int32),
            [pltpu.VMEM((EB, D), jnp.float32) for _ in range(NBUF)],
            pltpu.VMEM((320, D), jnp.float32),
            [pltpu.SemaphoreType.DMA for _ in range(NBUF)],
            [pltpu.SemaphoreType.DMA for _ in range(NBUF)],
            pltpu.VMEM_SHARED((NP, D), jnp.float32),
        ],
        compiler_params=pltpu.CompilerParams(use_tc_tiling_on_sc=False),
    )
    return kfn(eib, tbl)


DQ = D_H // 2


def _enc1_tc(dinv_ref, x_ref, w1_ref, b1_ref, t1_ref, lib_ref):
    dinv = dinv_ref[...]
    x = x_ref[...]
    p = jnp.dot(x, w1_ref[...], preferred_element_type=jnp.float32) + b1_ref[...]
    t1_ref[...] = p * dinv
    lib_ref[...] = jnp.sum(x, axis=1, keepdims=True)


def _enc1_call(dinv, x, W1, b1):
    grid = N // BR
    return pl.pallas_call(
        _enc1_tc,
        grid=(grid,),
        in_specs=[
            pl.BlockSpec((BR, 1), lambda i: (i, 0)),
            pl.BlockSpec((BR, D_IN), lambda i: (i, 0)),
            pl.BlockSpec((D_IN, D_H), lambda i: (0, 0)),
            pl.BlockSpec((1, D_H), lambda i: (0, 0)),
        ],
        out_specs=[
            pl.BlockSpec((BR, D_H), lambda i: (i, 0)),
            pl.BlockSpec((BR, 1), lambda i: (i, 0)),
        ],
        out_shape=[
            jax.ShapeDtypeStruct((N, D_H), jnp.float32),
            jax.ShapeDtypeStruct((N, 1), jnp.float32),
        ],
    )(dinv, x, W1, b1)


def _enc2_tc(s1a_ref, s1b_ref, t1_ref, dinv_ref, wc_ref, bc_ref, t2_ref):
    dinv = dinv_ref[...]
    t1 = t1_ref[...]
    sa = s1a_ref[0][:, :DQ] + s1a_ref[1][:, :DQ]
    sb = s1b_ref[0][:, :DQ] + s1b_ref[1][:, :DQ]
    ha = (sa + t1[:, :DQ]) * dinv
    hb = (sb + t1[:, DQ:]) * dinv
    h = jnp.maximum(jnp.concatenate([ha, hb], axis=1), 0.0)
    p = jnp.dot(h, wc_ref[...], preferred_element_type=jnp.float32) + bc_ref[...]
    t2_ref[...] = p * dinv


def _enc2_call(s1a, s1b, t1, dinv, Wc, bc):
    grid = N // BR
    return pl.pallas_call(
        _enc2_tc,
        grid=(grid,),
        in_specs=[
            pl.BlockSpec((NC, BR, EB), lambda i: (0, i, 0)),
            pl.BlockSpec((NC, BR, EB), lambda i: (0, i, 0)),
            pl.BlockSpec((BR, D_H), lambda i: (i, 0)),
            pl.BlockSpec((BR, 1), lambda i: (i, 0)),
            pl.BlockSpec((D_H, 2 * D_Z), lambda i: (0, 0)),
            pl.BlockSpec((1, 2 * D_Z), lambda i: (0, 0)),
        ],
        out_specs=pl.BlockSpec((BR, 2 * D_Z), lambda i: (i, 0)),
        out_shape=jax.ShapeDtypeStruct((N, 2 * D_Z), jnp.float32),
    )(s1a, s1b, t1, dinv, Wc, bc)


def _dec_tc(s2_ref, t2_ref, dinv_ref, eps_ref, lib_ref, wnb_ref, wzi_ref,
            mask_ref, mu_ref, logstd_ref, z_ref, nb_ref, zi_ref):
    s2 = s2_ref[0][:, :2 * D_Z] + s2_ref[1][:, :2 * D_Z]
    g = (s2 + t2_ref[...]) * dinv_ref[...]
    mu = g[:, :D_Z]
    logstd = g[:, D_Z:]
    z = mu + eps_ref[...] * jnp.exp(logstd)
    mu_ref[...] = mu
    logstd_ref[...] = logstd
    z_ref[...] = z
    wnb = wnb_ref[...] * mask_ref[...]
    wzi = wzi_ref[...] * mask_ref[...]
    nb_logits = jnp.dot(z, wnb, preferred_element_type=jnp.float32)
    m = jnp.max(nb_logits, axis=1, keepdims=True)
    ex = jnp.exp(nb_logits - m)
    nb_ref[...] = lib_ref[...] * ex / jnp.sum(ex, axis=1, keepdims=True)
    zi_ref[...] = jnp.dot(z, wzi, preferred_element_type=jnp.float32)


def _dec_call(s2, t2, dinv, eps, lib, W_nb, W_zi, mask):
    grid = N // BR
    return pl.pallas_call(
        _dec_tc,
        grid=(grid,),
        in_specs=[
            pl.BlockSpec((NC, BR, EB), lambda i: (0, i, 0)),
            pl.BlockSpec((BR, 2 * D_Z), lambda i: (i, 0)),
            pl.BlockSpec((BR, 1), lambda i: (i, 0)),
            pl.BlockSpec((BR, D_Z), lambda i: (i, 0)),
            pl.BlockSpec((BR, 1), lambda i: (i, 0)),
            pl.BlockSpec((D_Z, D_IN), lambda i: (0, 0)),
            pl.BlockSpec((D_Z, D_IN), lambda i: (0, 0)),
            pl.BlockSpec((D_Z, D_IN), lambda i: (0, 0)),
        ],
        out_specs=[
            pl.BlockSpec((BR, D_Z), lambda i: (i, 0)),
            pl.BlockSpec((BR, D_Z), lambda i: (i, 0)),
            pl.BlockSpec((BR, D_Z), lambda i: (i, 0)),
            pl.BlockSpec((BR, D_IN), lambda i: (i, 0)),
            pl.BlockSpec((BR, D_IN), lambda i: (i, 0)),
        ],
        out_shape=[
            jax.ShapeDtypeStruct((N, D_Z), jnp.float32),
            jax.ShapeDtypeStruct((N, D_Z), jnp.float32),
            jax.ShapeDtypeStruct((N, D_Z), jnp.float32),
            jax.ShapeDtypeStruct((N, D_IN), jnp.float32),
            jax.ShapeDtypeStruct((N, D_IN), jnp.float32),
        ],
    )(s2, t2, dinv, eps, lib, W_nb, W_zi, mask)


def _adj_tc(zr_ref, zc_ref, adj_ref):
    adj_ref[...] = lax.dot_general(
        zr_ref[...], zc_ref[...], (((1,), (1,)), ((), ())),
        preferred_element_type=jnp.float32)


def _adj_call(z):
    BRR = 400
    return pl.pallas_call(
        _adj_tc,
        grid=(N // BRR,),
        in_specs=[
            pl.BlockSpec((BRR, D_Z), lambda i: (i, 0)),
            pl.BlockSpec((N, D_Z), lambda i: (0, 0)),
        ],
        out_specs=pl.BlockSpec((BRR, N), lambda i: (i, 0)),
        out_shape=jax.ShapeDtypeStruct((N, N), jnp.float32),
    )(z, z)


def kernel(x, edge_index, W1, b1, W_mu, b_mu, W_logstd, b_logstd,
           W_nb, W_zi, mask, eps):
    eib = edge_index.reshape(2, NBLK, EB)

    deg = _deg_call(eib)
    dinv = lax.rsqrt(deg[0, :N] + deg[1, :N] + 1.0).reshape(N, 1)
    t1, lib = _enc1_call(dinv, x, W1, b1.reshape(1, D_H))
    t1v = t1.reshape(2 * N, DQ)
    s1a = _prop_call(eib, t1v, DQ, 2, 0)
    s1b = _prop_call(eib, t1v, DQ, 2, 1)
    Wc = jnp.concatenate([W_mu, W_logstd], axis=1)
    bc = jnp.concatenate([b_mu, b_logstd]).reshape(1, 2 * D_Z)
    t2 = _enc2_call(s1a, s1b, t1, dinv, Wc, bc)
    s2 = _prop_call(eib, t2, 2 * D_Z)
    mu, logstd, z, nb_means, zi = _dec_call(
        s2, t2, dinv, eps, lib, W_nb, W_zi, mask)
    adj = _adj_call(z)
    return (adj, nb_means, zi, mu, logstd)


# speedup vs baseline: 1.1298x; 1.0087x over previous
import functools

import jax
import jax.numpy as jnp
from jax import lax
from jax.experimental import pallas as pl
from jax.experimental.pallas import tpu as pltpu
from jax.experimental.pallas import tpu_sc as plsc

N = 10000
E = 320000
D_IN = 128
D_H = 128
D_Z = 32

NC = 2
NS = 16
NW = NC * NS
EB = 128
NB_W = 80
NBLK = E // EB
NB_LAST = NBLK - (NW - 1) * NB_W
NP = 10240
RPT = 640

BR = 2000


def _worker_id():
    return lax.axis_index("s") * NC + lax.axis_index("c")


def _deg_sc(eib, deg_out, didx, ones_v, tmpd, acc, sem):
    c = lax.axis_index("c")
    s = lax.axis_index("s")
    w = _worker_id()
    off = pl.multiple_of(s * RPT, 8)

    def zfill(i, carry):
        tmpd[pl.ds(i * 16, 16)] = jnp.zeros((16,), jnp.float32)
        return carry
    lax.fori_loop(0, RPT // 16, zfill, 0)
    for j in range(EB // 16):
        ones_v[pl.ds(j * 16, 16)] = jnp.ones((16,), jnp.float32)
    pltpu.sync_copy(tmpd, acc.at[pl.ds(off, RPT)])
    plsc.subcore_barrier()

    nb = jnp.where(w == NW - 1, NB_LAST, NB_W)

    @pl.when(w < NW - 1)
    def _():
        pltpu.sync_copy(eib.at[1, pl.ds(w * NB_W, NB_W)], didx)

    @pl.when(w == NW - 1)
    def _():
        pltpu.sync_copy(eib.at[1, pl.ds((NW - 1) * NB_W, NB_LAST)],
                        didx.at[pl.ds(0, NB_LAST)])

    def fire(i, carry):
        pltpu.async_copy(ones_v, acc.at[didx.at[i]], sem, add=True)
        return carry
    lax.fori_loop(0, nb, fire, 0)

    def drain(i, carry):
        pltpu.make_async_copy(ones_v, acc.at[didx.at[0]], sem).wait()
        return carry
    lax.fori_loop(0, nb, drain, 0)
    plsc.subcore_barrier()

    pltpu.sync_copy(acc.at[pl.ds(off, RPT)], tmpd)
    pltpu.sync_copy(tmpd, deg_out.at[c, pl.ds(off, RPT)])


def _deg_call(eib):
    kfn = pl.kernel(
        _deg_sc,
        out_type=jax.ShapeDtypeStruct((NC, NP), jnp.float32),
        mesh=plsc.VectorSubcoreMesh(
            core_axis_name="c", subcore_axis_name="s",
            num_cores=NC, num_subcores=NS),
        scratch_types=[
            pltpu.VMEM((NB_W, EB), jnp.int32),
            pltpu.VMEM((EB,), jnp.float32),
            pltpu.VMEM((RPT,), jnp.float32),
            pltpu.VMEM_SHARED((NP,), jnp.float32),
            pltpu.SemaphoreType.DMA,
        ],
        compiler_params=pltpu.CompilerParams(use_tc_tiling_on_sc=False),
    )
    return kfn(eib)


NBUF = 4


def _prop_sc(D, MULT, ADD, eib, tbl, out, sidx, didx, rows, tmp,
             gsems, ssems, acc):
    c = lax.axis_index("c")
    s = lax.axis_index("s")
    w = _worker_id()
    off = pl.multiple_of(s * RPT, 8)

    def zrow(r, carry):
        for j in range(D // 16):
            rows[0][r, pl.ds(j * 16, 16)] = jnp.zeros((16,), jnp.float32)
        return carry
    lax.fori_loop(0, EB, zrow, 0)
    for q in range(RPT // EB):
        pltpu.sync_copy(rows[0], acc.at[pl.ds(off + q * EB, EB)])
    plsc.subcore_barrier()

    nb = jnp.where(w == NW - 1, NB_LAST, NB_W)

    @pl.when(w < NW - 1)
    def _():
        pltpu.sync_copy(eib.at[0, pl.ds(w * NB_W, NB_W)], sidx)
        pltpu.sync_copy(eib.at[1, pl.ds(w * NB_W, NB_W)], didx)

    @pl.when(w == NW - 1)
    def _():
        pltpu.sync_copy(eib.at[0, pl.ds((NW - 1) * NB_W, NB_LAST)],
                        sidx.at[pl.ds(0, NB_LAST)])
        pltpu.sync_copy(eib.at[1, pl.ds((NW - 1) * NB_W, NB_LAST)],
                        didx.at[pl.ds(0, NB_LAST)])

    if MULT != 1 or ADD != 0:
        def xform(r, carry):
            for j in range(EB // 16):
                v = sidx[r, pl.ds(j * 16, 16)]
                sidx[r, pl.ds(j * 16, 16)] = v * MULT + ADD
            return carry
        lax.fori_loop(0, NB_W, xform, 0)

    def g_start(i, j):
        pltpu.async_copy(tbl.at[sidx.at[i]], rows[j], gsems[j])

    def g_wait(i, j):
        pltpu.make_async_copy(tbl.at[sidx.at[i]], rows[j], gsems[j]).wait()

    def s_start(i, j):
        pltpu.async_copy(rows[j], acc.at[didx.at[i]], ssems[j], add=True)

    def s_wait(i, j):
        pltpu.make_async_copy(rows[j], acc.at[didx.at[i]], ssems[j]).wait()

    for j in range(NBUF):
        g_start(j, j)

    def body(k, carry):
        base = NBUF * k
        for j in range(NBUF):
            g_wait(base + j, j)
            s_start(base + j, j)
        for j in range(NBUF):
            s_wait(base + j, j)

            @pl.when(base + NBUF + j < nb)
            def _():
                g_start(base + NBUF + j, j)
        return carry
    lax.fori_loop(0, nb // NBUF, body, 0)
    plsc.subcore_barrier()

    CH = 320
    for q in range(RPT // CH):
        pltpu.sync_copy(acc.at[pl.ds(off + q * CH, CH)], tmp)
        pltpu.sync_copy(tmp, out.at[c, pl.ds(off + q * CH, CH), pl.ds(0, D)])


def _prop_call(eib, tbl, D, mult=1, add=0):
    kfn = pl.kernel(
        functools.partial(_prop_sc, D, mult, add),
        out_type=jax.ShapeDtypeStruct((NC, NP, EB), jnp.float32),
        mesh=plsc.VectorSubcoreMesh(
            core_axis_name="c", subcore_axis_name="s",
            num_cores=NC, num_subcores=NS),
        scratch_types=[
            pltpu.VMEM((NB_W, EB), jnp.int32),
            pltpu.VMEM((NB_W, EB), jnp.int32),
            [pltpu.VMEM((EB, D), jnp.float32) for _ in range(NBUF)],
            pltpu.VMEM((320, D), jnp.float32),
            [pltpu.SemaphoreType.DMA for _ in range(NBUF)],
            [pltpu.SemaphoreType.DMA for _ in range(NBUF)],
            pltpu.VMEM_SHARED((NP, D), jnp.float32),
        ],
        compiler_params=pltpu.CompilerParams(use_tc_tiling_on_sc=False),
    )
    return kfn(eib, tbl)


DQ = D_H // 2


def _enc1_tc(dinv_ref, x_ref, w1_ref, b1_ref, t1_ref, lib_ref):
    dinv = dinv_ref[...]
    x = x_ref[...]
    p = jnp.dot(x, w1_ref[...], preferred_element_type=jnp.float32) + b1_ref[...]
    t1_ref[...] = p * dinv
    lib_ref[...] = jnp.sum(x, axis=1, keepdims=True)


def _enc1_call(dinv, x, W1, b1):
    grid = N // BR
    return pl.pallas_call(
        _enc1_tc,
        grid=(grid,),
        in_specs=[
            pl.BlockSpec((BR, 1), lambda i: (i, 0)),
            pl.BlockSpec((BR, D_IN), lambda i: (i, 0)),
            pl.BlockSpec((D_IN, D_H), lambda i: (0, 0)),
            pl.BlockSpec((1, D_H), lambda i: (0, 0)),
        ],
        out_specs=[
            pl.BlockSpec((BR, D_H), lambda i: (i, 0)),
            pl.BlockSpec((BR, 1), lambda i: (i, 0)),
        ],
        out_shape=[
            jax.ShapeDtypeStruct((N, D_H), jnp.float32),
            jax.ShapeDtypeStruct((N, 1), jnp.float32),
        ],
    )(dinv, x, W1, b1)


def _enc2_tc(s1a_ref, s1b_ref, t1_ref, dinv_ref, wc_ref, bc_ref, t2_ref):
    dinv = dinv_ref[...]
    t1 = t1_ref[...]
    sa = s1a_ref[0][:, :DQ] + s1a_ref[1][:, :DQ]
    sb = s1b_ref[0][:, :DQ] + s1b_ref[1][:, :DQ]
    ha = (sa + t1[:, :DQ]) * dinv
    hb = (sb + t1[:, DQ:]) * dinv
    h = jnp.maximum(jnp.concatenate([ha, hb], axis=1), 0.0)
    p = jnp.dot(h, wc_ref[...], preferred_element_type=jnp.float32) + bc_ref[...]
    t2 = p * dinv
    t2_ref[...] = jnp.concatenate([t2, t2], axis=1)


def _enc2_call(s1a, s1b, t1, dinv, Wc, bc):
    grid = N // BR
    return pl.pallas_call(
        _enc2_tc,
        grid=(grid,),
        in_specs=[
            pl.BlockSpec((NC, BR, EB), lambda i: (0, i, 0)),
            pl.BlockSpec((NC, BR, EB), lambda i: (0, i, 0)),
            pl.BlockSpec((BR, D_H), lambda i: (i, 0)),
            pl.BlockSpec((BR, 1), lambda i: (i, 0)),
            pl.BlockSpec((D_H, 2 * D_Z), lambda i: (0, 0)),
            pl.BlockSpec((1, 2 * D_Z), lambda i: (0, 0)),
        ],
        out_specs=pl.BlockSpec((BR, D_H), lambda i: (i, 0)),
        out_shape=jax.ShapeDtypeStruct((N, D_H), jnp.float32),
    )(s1a, s1b, t1, dinv, Wc, bc)


def _dec_tc(s2_ref, t2_ref, dinv_ref, eps_ref, lib_ref, wnb_ref, wzi_ref,
            mask_ref, mu_ref, logstd_ref, z_ref, z2_ref, nb_ref, zi_ref):
    s2 = s2_ref[0][:, :2 * D_Z] + s2_ref[1][:, :2 * D_Z]
    g = (s2 + t2_ref[...][:, :2 * D_Z]) * dinv_ref[...]
    mu = g[:, :D_Z]
    logstd = g[:, D_Z:]
    z = mu + eps_ref[...] * jnp.exp(logstd)
    mu_ref[...] = mu
    logstd_ref[...] = logstd
    z_ref[...] = z
    z2_ref[...] = z
    wnb = wnb_ref[...] * mask_ref[...]
    wzi = wzi_ref[...] * mask_ref[...]
    nb_logits = jnp.dot(z, wnb, preferred_element_type=jnp.float32)
    m = jnp.max(nb_logits, axis=1, keepdims=True)
    ex = jnp.exp(nb_logits - m)
    nb_ref[...] = lib_ref[...] * ex / jnp.sum(ex, axis=1, keepdims=True)
    zi_ref[...] = jnp.dot(z, wzi, preferred_element_type=jnp.float32)


def _dec_call(s2, t2, dinv, eps, lib, W_nb, W_zi, mask):
    grid = N // BR
    return pl.pallas_call(
        _dec_tc,
        grid=(grid,),
        in_specs=[
            pl.BlockSpec((NC, BR, EB), lambda i: (0, i, 0)),
            pl.BlockSpec((BR, D_H), lambda i: (i, 0)),
            pl.BlockSpec((BR, 1), lambda i: (i, 0)),
            pl.BlockSpec((BR, D_Z), lambda i: (i, 0)),
            pl.BlockSpec((BR, 1), lambda i: (i, 0)),
            pl.BlockSpec((D_Z, D_IN), lambda i: (0, 0)),
            pl.BlockSpec((D_Z, D_IN), lambda i: (0, 0)),
            pl.BlockSpec((D_Z, D_IN), lambda i: (0, 0)),
        ],
        out_specs=[
            pl.BlockSpec((BR, D_Z), lambda i: (i, 0)),
            pl.BlockSpec((BR, D_Z), lambda i: (i, 0)),
            pl.BlockSpec((BR, D_Z), lambda i: (i, 0)),
            pl.BlockSpec((BR, D_Z), lambda i: (i, 0)),
            pl.BlockSpec((BR, D_IN), lambda i: (i, 0)),
            pl.BlockSpec((BR, D_IN), lambda i: (i, 0)),
        ],
        out_shape=[
            jax.ShapeDtypeStruct((N, D_Z), jnp.float32),
            jax.ShapeDtypeStruct((N, D_Z), jnp.float32),
            jax.ShapeDtypeStruct((N, D_Z), jnp.float32),
            jax.ShapeDtypeStruct((N, D_Z), jnp.float32),
            jax.ShapeDtypeStruct((N, D_IN), jnp.float32),
            jax.ShapeDtypeStruct((N, D_IN), jnp.float32),
        ],
    )(s2, t2, dinv, eps, lib, W_nb, W_zi, mask)


def _adj_tc(zr_ref, zc_ref, adj_ref):
    adj_ref[...] = lax.dot_general(
        zr_ref[...], zc_ref[...], (((1,), (1,)), ((), ())),
        preferred_element_type=jnp.float32)


def _adj_call(z, z2):
    BRR = 400
    return pl.pallas_call(
        _adj_tc,
        grid=(N // BRR,),
        in_specs=[
            pl.BlockSpec((BRR, D_Z), lambda i: (i, 0)),
            pl.BlockSpec((N, D_Z), lambda i: (0, 0)),
        ],
        out_specs=pl.BlockSpec((BRR, N), lambda i: (i, 0)),
        out_shape=jax.ShapeDtypeStruct((N, N), jnp.float32),
    )(z, z2)


def kernel(x, edge_index, W1, b1, W_mu, b_mu, W_logstd, b_logstd,
           W_nb, W_zi, mask, eps):
    eib = edge_index.reshape(2, NBLK, EB)

    deg = _deg_call(eib)
    dinv = lax.rsqrt(deg[0, :N] + deg[1, :N] + 1.0).reshape(N, 1)
    t1, lib = _enc1_call(dinv, x, W1, b1.reshape(1, D_H))
    t1v = t1.reshape(2 * N, DQ)
    s1a = _prop_call(eib, t1v, DQ, 2, 0)
    s1b = _prop_call(eib, t1v, DQ, 2, 1)
    Wc = jnp.concatenate([W_mu, W_logstd], axis=1)
    bc = jnp.concatenate([b_mu, b_logstd]).reshape(1, 2 * D_Z)
    t2 = _enc2_call(s1a, s1b, t1, dinv, Wc, bc)
    t2v = t2.reshape(2 * N, 2 * D_Z)
    s2 = _prop_call(eib, t2v, 2 * D_Z, 2, 0)
    mu, logstd, z, z2, nb_means, zi = _dec_call(
        s2, t2, dinv, eps, lib, W_nb, W_zi, mask)
    adj = _adj_call(z, z2)
    return (adj, nb_means, zi, mu, logstd)


# speedup vs baseline: 1.1392x; 1.0083x over previous
import functools

import jax
import jax.numpy as jnp
from jax import lax
from jax.experimental import pallas as pl
from jax.experimental.pallas import tpu as pltpu
from jax.experimental.pallas import tpu_sc as plsc

N = 10000
E = 320000
D_IN = 128
D_H = 128
D_Z = 32

NC = 2
NS = 16
NW = NC * NS
EB = 128
NB_W = 80
NBLK = E // EB
NB_LAST = NBLK - (NW - 1) * NB_W
NP = 10240
RPT = 640

BR = 2000


def _worker_id():
    return lax.axis_index("s") * NC + lax.axis_index("c")


def _deg_sc(eib, deg_out, didx, ones_v, tmpd, acc, sem):
    c = lax.axis_index("c")
    s = lax.axis_index("s")
    w = _worker_id()
    off = pl.multiple_of(s * RPT, 8)

    def zfill(i, carry):
        tmpd[pl.ds(i * 16, 16)] = jnp.zeros((16,), jnp.float32)
        return carry
    lax.fori_loop(0, RPT // 16, zfill, 0)
    for j in range(EB // 16):
        ones_v[pl.ds(j * 16, 16)] = jnp.ones((16,), jnp.float32)
    pltpu.sync_copy(tmpd, acc.at[pl.ds(off, RPT)])
    plsc.subcore_barrier()

    nb = jnp.where(w == NW - 1, NB_LAST, NB_W)

    @pl.when(w < NW - 1)
    def _():
        pltpu.sync_copy(eib.at[1, pl.ds(w * NB_W, NB_W)], didx)

    @pl.when(w == NW - 1)
    def _():
        pltpu.sync_copy(eib.at[1, pl.ds((NW - 1) * NB_W, NB_LAST)],
                        didx.at[pl.ds(0, NB_LAST)])

    def fire(i, carry):
        pltpu.async_copy(ones_v, acc.at[didx.at[i]], sem, add=True)
        return carry
    lax.fori_loop(0, nb, fire, 0)

    def drain(i, carry):
        pltpu.make_async_copy(ones_v, acc.at[didx.at[0]], sem).wait()
        return carry
    lax.fori_loop(0, nb, drain, 0)
    plsc.subcore_barrier()

    pltpu.sync_copy(acc.at[pl.ds(off, RPT)], tmpd)
    pltpu.sync_copy(tmpd, deg_out.at[c, pl.ds(off, RPT)])


def _deg_call(eib):
    kfn = pl.kernel(
        _deg_sc,
        out_type=jax.ShapeDtypeStruct((NC, NP), jnp.float32),
        mesh=plsc.VectorSubcoreMesh(
            core_axis_name="c", subcore_axis_name="s",
            num_cores=NC, num_subcores=NS),
        scratch_types=[
            pltpu.VMEM((NB_W, EB), jnp.int32),
            pltpu.VMEM((EB,), jnp.float32),
            pltpu.VMEM((RPT,), jnp.float32),
            pltpu.VMEM_SHARED((NP,), jnp.float32),
            pltpu.SemaphoreType.DMA,
        ],
        compiler_params=pltpu.CompilerParams(use_tc_tiling_on_sc=False),
    )
    return kfn(eib)


NBUF = 5


def _prop_sc(D, MULT, ADD, eib, tbl, out, sidx, didx, rows, tmp,
             gsems, ssems, acc):
    c = lax.axis_index("c")
    s = lax.axis_index("s")
    w = _worker_id()
    off = pl.multiple_of(s * RPT, 8)

    def zrow(r, carry):
        for j in range(D // 16):
            rows[0][r, pl.ds(j * 16, 16)] = jnp.zeros((16,), jnp.float32)
        return carry
    lax.fori_loop(0, EB, zrow, 0)
    for q in range(RPT // EB):
        pltpu.sync_copy(rows[0], acc.at[pl.ds(off + q * EB, EB)])
    plsc.subcore_barrier()

    nb = jnp.where(w == NW - 1, NB_LAST, NB_W)

    @pl.when(w < NW - 1)
    def _():
        pltpu.sync_copy(eib.at[0, pl.ds(w * NB_W, NB_W)], sidx)
        pltpu.sync_copy(eib.at[1, pl.ds(w * NB_W, NB_W)], didx)

    @pl.when(w == NW - 1)
    def _():
        pltpu.sync_copy(eib.at[0, pl.ds((NW - 1) * NB_W, NB_LAST)],
                        sidx.at[pl.ds(0, NB_LAST)])
        pltpu.sync_copy(eib.at[1, pl.ds((NW - 1) * NB_W, NB_LAST)],
                        didx.at[pl.ds(0, NB_LAST)])

    if MULT != 1 or ADD != 0:
        def xform(r, carry):
            for j in range(EB // 16):
                v = sidx[r, pl.ds(j * 16, 16)]
                sidx[r, pl.ds(j * 16, 16)] = v * MULT + ADD
            return carry
        lax.fori_loop(0, NB_W, xform, 0)

    def g_start(i, j):
        pltpu.async_copy(tbl.at[sidx.at[i]], rows[j], gsems[j])

    def g_wait(i, j):
        pltpu.make_async_copy(tbl.at[sidx.at[i]], rows[j], gsems[j]).wait()

    def s_start(i, j):
        pltpu.async_copy(rows[j], acc.at[didx.at[i]], ssems[j], add=True)

    def s_wait(i, j):
        pltpu.make_async_copy(rows[j], acc.at[didx.at[i]], ssems[j]).wait()

    for j in range(NBUF):
        g_start(j, j)

    def body(k, carry):
        base = NBUF * k
        for j in range(NBUF):
            g_wait(base + j, j)
            s_start(base + j, j)
        for j in range(NBUF):
            s_wait(base + j, j)

            @pl.when(base + NBUF + j < nb)
            def _():
                g_start(base + NBUF + j, j)
        return carry
    lax.fori_loop(0, nb // NBUF, body, 0)
    plsc.subcore_barrier()

    CH = 320
    for q in range(RPT // CH):
        pltpu.sync_copy(acc.at[pl.ds(off + q * CH, CH)], tmp)
        pltpu.sync_copy(tmp, out.at[c, pl.ds(off + q * CH, CH), pl.ds(0, D)])


def _prop_call(eib, tbl, D, mult=1, add=0):
    kfn = pl.kernel(
        functools.partial(_prop_sc, D, mult, add),
        out_type=jax.ShapeDtypeStruct((NC, NP, EB), jnp.float32),
        mesh=plsc.VectorSubcoreMesh(
            core_axis_name="c", subcore_axis_name="s",
            num_cores=NC, num_subcores=NS),
        scratch_types=[
            pltpu.VMEM((NB_W, EB), jnp.int32),
            pltpu.VMEM((NB_W, EB), jnp.int32),
            [pltpu.VMEM((EB, D), jnp.float32) for _ in range(NBUF)],
            pltpu.VMEM((320, D), jnp.float32),
            [pltpu.SemaphoreType.DMA for _ in range(NBUF)],
            [pltpu.SemaphoreType.DMA for _ in range(NBUF)],
            pltpu.VMEM_SHARED((NP, D), jnp.float32),
        ],
        compiler_params=pltpu.CompilerParams(use_tc_tiling_on_sc=False),
    )
    return kfn(eib, tbl)


DQ = D_H // 2


def _enc1_tc(dinv_ref, x_ref, w1_ref, b1_ref, t1_ref, lib_ref):
    dinv = dinv_ref[...]
    x = x_ref[...]
    p = jnp.dot(x, w1_ref[...], preferred_element_type=jnp.float32) + b1_ref[...]
    t1_ref[...] = p * dinv
    lib_ref[...] = jnp.sum(x, axis=1, keepdims=True)


def _enc1_call(dinv, x, W1, b1):
    grid = N // BR
    return pl.pallas_call(
        _enc1_tc,
        grid=(grid,),
        in_specs=[
            pl.BlockSpec((BR, 1), lambda i: (i, 0)),
            pl.BlockSpec((BR, D_IN), lambda i: (i, 0)),
            pl.BlockSpec((D_IN, D_H), lambda i: (0, 0)),
            pl.BlockSpec((1, D_H), lambda i: (0, 0)),
        ],
        out_specs=[
            pl.BlockSpec((BR, D_H), lambda i: (i, 0)),
            pl.BlockSpec((BR, 1), lambda i: (i, 0)),
        ],
        out_shape=[
            jax.ShapeDtypeStruct((N, D_H), jnp.float32),
            jax.ShapeDtypeStruct((N, 1), jnp.float32),
        ],
    )(dinv, x, W1, b1)


def _enc2_tc(s1a_ref, s1b_ref, t1_ref, dinv_ref, wc_ref, bc_ref, t2_ref):
    dinv = dinv_ref[...]
    t1 = t1_ref[...]
    sa = s1a_ref[0][:, :DQ] + s1a_ref[1][:, :DQ]
    sb = s1b_ref[0][:, :DQ] + s1b_ref[1][:, :DQ]
    ha = (sa + t1[:, :DQ]) * dinv
    hb = (sb + t1[:, DQ:]) * dinv
    h = jnp.maximum(jnp.concatenate([ha, hb], axis=1), 0.0)
    p = jnp.dot(h, wc_ref[...], preferred_element_type=jnp.float32) + bc_ref[...]
    t2 = p * dinv
    t2_ref[...] = jnp.concatenate([t2, t2], axis=1)


def _enc2_call(s1a, s1b, t1, dinv, Wc, bc):
    grid = N // BR
    return pl.pallas_call(
        _enc2_tc,
        grid=(grid,),
        in_specs=[
            pl.BlockSpec((NC, BR, EB), lambda i: (0, i, 0)),
            pl.BlockSpec((NC, BR, EB), lambda i: (0, i, 0)),
            pl.BlockSpec((BR, D_H), lambda i: (i, 0)),
            pl.BlockSpec((BR, 1), lambda i: (i, 0)),
            pl.BlockSpec((D_H, 2 * D_Z), lambda i: (0, 0)),
            pl.BlockSpec((1, 2 * D_Z), lambda i: (0, 0)),
        ],
        out_specs=pl.BlockSpec((BR, D_H), lambda i: (i, 0)),
        out_shape=jax.ShapeDtypeStruct((N, D_H), jnp.float32),
    )(s1a, s1b, t1, dinv, Wc, bc)


def _dec_tc(s2_ref, t2_ref, dinv_ref, eps_ref, lib_ref, wnb_ref, wzi_ref,
            mask_ref, mu_ref, logstd_ref, z_ref, z2_ref, nb_ref, zi_ref):
    s2 = s2_ref[0][:, :2 * D_Z] + s2_ref[1][:, :2 * D_Z]
    g = (s2 + t2_ref[...][:, :2 * D_Z]) * dinv_ref[...]
    mu = g[:, :D_Z]
    logstd = g[:, D_Z:]
    z = mu + eps_ref[...] * jnp.exp(logstd)
    mu_ref[...] = mu
    logstd_ref[...] = logstd
    z_ref[...] = z
    z2_ref[...] = z
    wnb = wnb_ref[...] * mask_ref[...]
    wzi = wzi_ref[...] * mask_ref[...]
    nb_logits = jnp.dot(z, wnb, preferred_element_type=jnp.float32)
    m = jnp.max(nb_logits, axis=1, keepdims=True)
    ex = jnp.exp(nb_logits - m)
    nb_ref[...] = lib_ref[...] * ex / jnp.sum(ex, axis=1, keepdims=True)
    zi_ref[...] = jnp.dot(z, wzi, preferred_element_type=jnp.float32)


def _dec_call(s2, t2, dinv, eps, lib, W_nb, W_zi, mask):
    grid = N // BR
    return pl.pallas_call(
        _dec_tc,
        grid=(grid,),
        in_specs=[
            pl.BlockSpec((NC, BR, EB), lambda i: (0, i, 0)),
            pl.BlockSpec((BR, D_H), lambda i: (i, 0)),
            pl.BlockSpec((BR, 1), lambda i: (i, 0)),
            pl.BlockSpec((BR, D_Z), lambda i: (i, 0)),
            pl.BlockSpec((BR, 1), lambda i: (i, 0)),
            pl.BlockSpec((D_Z, D_IN), lambda i: (0, 0)),
            pl.BlockSpec((D_Z, D_IN), lambda i: (0, 0)),
            pl.BlockSpec((D_Z, D_IN), lambda i: (0, 0)),
        ],
        out_specs=[
            pl.BlockSpec((BR, D_Z), lambda i: (i, 0)),
            pl.BlockSpec((BR, D_Z), lambda i: (i, 0)),
            pl.BlockSpec((BR, D_Z), lambda i: (i, 0)),
            pl.BlockSpec((BR, D_Z), lambda i: (i, 0)),
            pl.BlockSpec((BR, D_IN), lambda i: (i, 0)),
            pl.BlockSpec((BR, D_IN), lambda i: (i, 0)),
        ],
        out_shape=[
            jax.ShapeDtypeStruct((N, D_Z), jnp.float32),
            jax.ShapeDtypeStruct((N, D_Z), jnp.float32),
            jax.ShapeDtypeStruct((N, D_Z), jnp.float32),
            jax.ShapeDtypeStruct((N, D_Z), jnp.float32),
            jax.ShapeDtypeStruct((N, D_IN), jnp.float32),
            jax.ShapeDtypeStruct((N, D_IN), jnp.float32),
        ],
    )(s2, t2, dinv, eps, lib, W_nb, W_zi, mask)


def _adj_tc(zr_ref, zc_ref, adj_ref):
    adj_ref[...] = lax.dot_general(
        zr_ref[...], zc_ref[...], (((1,), (1,)), ((), ())),
        preferred_element_type=jnp.float32)


def _adj_call(z, z2):
    BRR = 400
    return pl.pallas_call(
        _adj_tc,
        grid=(N // BRR,),
        in_specs=[
            pl.BlockSpec((BRR, D_Z), lambda i: (i, 0)),
            pl.BlockSpec((N, D_Z), lambda i: (0, 0)),
        ],
        out_specs=pl.BlockSpec((BRR, N), lambda i: (i, 0)),
        out_shape=jax.ShapeDtypeStruct((N, N), jnp.float32),
    )(z, z2)


def kernel(x, edge_index, W1, b1, W_mu, b_mu, W_logstd, b_logstd,
           W_nb, W_zi, mask, eps):
    eib = edge_index.reshape(2, NBLK, EB)

    deg = _deg_call(eib)
    dinv = lax.rsqrt(deg[0, :N] + deg[1, :N] + 1.0).reshape(N, 1)
    t1, lib = _enc1_call(dinv, x, W1, b1.reshape(1, D_H))
    t1v = t1.reshape(2 * N, DQ)
    s1a = _prop_call(eib, t1v, DQ, 2, 0)
    s1b = _prop_call(eib, t1v, DQ, 2, 1)
    Wc = jnp.concatenate([W_mu, W_logstd], axis=1)
    bc = jnp.concatenate([b_mu, b_logstd]).reshape(1, 2 * D_Z)
    t2 = _enc2_call(s1a, s1b, t1, dinv, Wc, bc)
    t2v = t2.reshape(2 * N, 2 * D_Z)
    s2 = _prop_call(eib, t2v, 2 * D_Z, 2, 0)
    mu, logstd, z, z2, nb_means, zi = _dec_call(
        s2, t2, dinv, eps, lib, W_nb, W_zi, mask)
    adj = _adj_call(z, z2)
    return (adj, nb_means, zi, mu, logstd)


# speedup vs baseline: 1.1444x; 1.0046x over previous
import functools

import jax
import jax.numpy as jnp
from jax import lax
from jax.experimental import pallas as pl
from jax.experimental.pallas import tpu as pltpu
from jax.experimental.pallas import tpu_sc as plsc

N = 10000
E = 320000
D_IN = 128
D_H = 128
D_Z = 32

NC = 2
NS = 16
NW = NC * NS
EB = 128
NB_W = 80
NBLK = E // EB
NB_LAST = NBLK - (NW - 1) * NB_W
NP = 10240
RPT = 640

BR = 2000


def _worker_id():
    return lax.axis_index("s") * NC + lax.axis_index("c")


def _deg_sc(eib, deg_out, didx, ones_v, tmpd, acc, sem):
    c = lax.axis_index("c")
    s = lax.axis_index("s")
    w = _worker_id()
    off = pl.multiple_of(s * RPT, 8)

    def zfill(i, carry):
        tmpd[pl.ds(i * 16, 16)] = jnp.zeros((16,), jnp.float32)
        return carry
    lax.fori_loop(0, RPT // 16, zfill, 0)
    for j in range(EB // 16):
        ones_v[pl.ds(j * 16, 16)] = jnp.ones((16,), jnp.float32)
    pltpu.sync_copy(tmpd, acc.at[pl.ds(off, RPT)])
    plsc.subcore_barrier()

    nb = jnp.where(w == NW - 1, NB_LAST, NB_W)

    @pl.when(w < NW - 1)
    def _():
        pltpu.sync_copy(eib.at[1, pl.ds(w * NB_W, NB_W)], didx)

    @pl.when(w == NW - 1)
    def _():
        pltpu.sync_copy(eib.at[1, pl.ds((NW - 1) * NB_W, NB_LAST)],
                        didx.at[pl.ds(0, NB_LAST)])

    def fire(i, carry):
        pltpu.async_copy(ones_v, acc.at[didx.at[i]], sem, add=True)
        return carry
    lax.fori_loop(0, nb, fire, 0)

    def drain(i, carry):
        pltpu.make_async_copy(ones_v, acc.at[didx.at[0]], sem).wait()
        return carry
    lax.fori_loop(0, nb, drain, 0)
    plsc.subcore_barrier()

    pltpu.sync_copy(acc.at[pl.ds(off, RPT)], tmpd)
    pltpu.sync_copy(tmpd, deg_out.at[c, pl.ds(off, RPT)])


def _deg_call(eib):
    kfn = pl.kernel(
        _deg_sc,
        out_type=jax.ShapeDtypeStruct((NC, NP), jnp.float32),
        mesh=plsc.VectorSubcoreMesh(
            core_axis_name="c", subcore_axis_name="s",
            num_cores=NC, num_subcores=NS),
        scratch_types=[
            pltpu.VMEM((NB_W, EB), jnp.int32),
            pltpu.VMEM((EB,), jnp.float32),
            pltpu.VMEM((RPT,), jnp.float32),
            pltpu.VMEM_SHARED((NP,), jnp.float32),
            pltpu.SemaphoreType.DMA,
        ],
        compiler_params=pltpu.CompilerParams(use_tc_tiling_on_sc=False),
    )
    return kfn(eib)


NBUF = 5


def _prop_sc(D, MULT, ADD, eib, tbl, out, sidx, didx, rows, tmp,
             gsems, ssems, acc):
    c = lax.axis_index("c")
    s = lax.axis_index("s")
    w = _worker_id()
    off = pl.multiple_of(s * RPT, 8)

    def zrow(r, carry):
        for j in range(D // 16):
            rows[0][r, pl.ds(j * 16, 16)] = jnp.zeros((16,), jnp.float32)
        return carry
    lax.fori_loop(0, EB, zrow, 0)
    for q in range(RPT // EB):
        pltpu.sync_copy(rows[0], acc.at[pl.ds(off + q * EB, EB)])
    plsc.subcore_barrier()

    nb = jnp.where(w == NW - 1, NB_LAST, NB_W)

    @pl.when(w < NW - 1)
    def _():
        pltpu.sync_copy(eib.at[0, pl.ds(w * NB_W, NB_W)], sidx)
        pltpu.sync_copy(eib.at[1, pl.ds(w * NB_W, NB_W)], didx)

    @pl.when(w == NW - 1)
    def _():
        pltpu.sync_copy(eib.at[0, pl.ds((NW - 1) * NB_W, NB_LAST)],
                        sidx.at[pl.ds(0, NB_LAST)])
        pltpu.sync_copy(eib.at[1, pl.ds((NW - 1) * NB_W, NB_LAST)],
                        didx.at[pl.ds(0, NB_LAST)])

    if MULT != 1 or ADD != 0:
        def xform(r, carry):
            for j in range(EB // 16):
                v = sidx[r, pl.ds(j * 16, 16)]
                sidx[r, pl.ds(j * 16, 16)] = v * MULT + ADD
            return carry
        lax.fori_loop(0, NB_W, xform, 0)

    def g_start(i, j):
        pltpu.async_copy(tbl.at[sidx.at[i]], rows[j], gsems[j])

    def g_wait(i, j):
        pltpu.make_async_copy(tbl.at[sidx.at[i]], rows[j], gsems[j]).wait()

    def s_start(i, j):
        pltpu.async_copy(rows[j], acc.at[didx.at[i]], ssems[j], add=True)

    def s_wait(i, j):
        pltpu.make_async_copy(rows[j], acc.at[didx.at[i]], ssems[j]).wait()

    for j in range(NBUF):
        g_start(j, j)

    def body(k, carry):
        base = NBUF * k
        for j in range(NBUF):
            g_wait(base + j, j)
            s_start(base + j, j)
        for j in range(NBUF):
            s_wait(base + j, j)

            @pl.when(base + NBUF + j < nb)
            def _():
                g_start(base + NBUF + j, j)
        return carry
    lax.fori_loop(0, nb // NBUF, body, 0)
    plsc.subcore_barrier()

    CH = 320
    for q in range(RPT // CH):
        pltpu.sync_copy(acc.at[pl.ds(off + q * CH, CH)], tmp)
        pltpu.sync_copy(tmp, out.at[c, pl.ds(off + q * CH, CH), pl.ds(0, D)])


def _prop_call(eib, tbl, D, mult=1, add=0):
    kfn = pl.kernel(
        functools.partial(_prop_sc, D, mult, add),
        out_type=jax.ShapeDtypeStruct((NC, NP, EB), jnp.float32),
        mesh=plsc.VectorSubcoreMesh(
            core_axis_name="c", subcore_axis_name="s",
            num_cores=NC, num_subcores=NS),
        scratch_types=[
            pltpu.VMEM((NB_W, EB), jnp.int32),
            pltpu.VMEM((NB_W, EB), jnp.int32),
            [pltpu.VMEM((EB, D), jnp.float32) for _ in range(NBUF)],
            pltpu.VMEM((320, D), jnp.float32),
            [pltpu.SemaphoreType.DMA for _ in range(NBUF)],
            [pltpu.SemaphoreType.DMA for _ in range(NBUF)],
            pltpu.VMEM_SHARED((NP, D), jnp.float32),
        ],
        compiler_params=pltpu.CompilerParams(use_tc_tiling_on_sc=False),
    )
    return kfn(eib, tbl)


DQ = D_H // 2


def _enc1_tc(dinv_ref, x_ref, w1_ref, b1_ref, t1_ref, lib_ref):
    dinv = dinv_ref[...]
    x = x_ref[...]
    p = jnp.dot(x, w1_ref[...], preferred_element_type=jnp.float32) + b1_ref[...]
    t1_ref[...] = p * dinv
    lib_ref[...] = jnp.sum(x, axis=1, keepdims=True)


def _enc1_call(dinv, x, W1, b1):
    grid = N // BR
    return pl.pallas_call(
        _enc1_tc,
        grid=(grid,),
        in_specs=[
            pl.BlockSpec((BR, 1), lambda i: (i, 0)),
            pl.BlockSpec((BR, D_IN), lambda i: (i, 0)),
            pl.BlockSpec((D_IN, D_H), lambda i: (0, 0)),
            pl.BlockSpec((1, D_H), lambda i: (0, 0)),
        ],
        out_specs=[
            pl.BlockSpec((BR, D_H), lambda i: (i, 0)),
            pl.BlockSpec((BR, 1), lambda i: (i, 0)),
        ],
        out_shape=[
            jax.ShapeDtypeStruct((N, D_H), jnp.float32),
            jax.ShapeDtypeStruct((N, 1), jnp.float32),
        ],
    )(dinv, x, W1, b1)


def _enc2_tc(s1a_ref, s1b_ref, t1_ref, dinv_ref, wc_ref, bc_ref, t2_ref):
    dinv = dinv_ref[...]
    t1 = t1_ref[...]
    sa = s1a_ref[0][:, :DQ] + s1a_ref[1][:, :DQ]
    sb = s1b_ref[0][:, :DQ] + s1b_ref[1][:, :DQ]
    ha = (sa + t1[:, :DQ]) * dinv
    hb = (sb + t1[:, DQ:]) * dinv
    h = jnp.maximum(jnp.concatenate([ha, hb], axis=1), 0.0)
    p = jnp.dot(h, wc_ref[...], preferred_element_type=jnp.float32) + bc_ref[...]
    t2 = p * dinv
    t2_ref[...] = jnp.concatenate([t2, t2], axis=1)


def _enc2_call(s1a, s1b, t1, dinv, Wc, bc):
    grid = N // BR
    return pl.pallas_call(
        _enc2_tc,
        grid=(grid,),
        in_specs=[
            pl.BlockSpec((NC, BR, EB), lambda i: (0, i, 0)),
            pl.BlockSpec((NC, BR, EB), lambda i: (0, i, 0)),
            pl.BlockSpec((BR, D_H), lambda i: (i, 0)),
            pl.BlockSpec((BR, 1), lambda i: (i, 0)),
            pl.BlockSpec((D_H, 2 * D_Z), lambda i: (0, 0)),
            pl.BlockSpec((1, 2 * D_Z), lambda i: (0, 0)),
        ],
        out_specs=pl.BlockSpec((BR, D_H), lambda i: (i, 0)),
        out_shape=jax.ShapeDtypeStruct((N, D_H), jnp.float32),
    )(s1a, s1b, t1, dinv, Wc, bc)


def _dec_tc(s2_ref, t2_ref, dinv_ref, eps_ref, lib_ref, wnb_ref, wzi_ref,
            mask_ref, mu_ref, logstd_ref, z_ref, z2_ref, nb_ref, zi_ref):
    s2 = s2_ref[0][:, :2 * D_Z] + s2_ref[1][:, :2 * D_Z]
    g = (s2 + t2_ref[...][:, :2 * D_Z]) * dinv_ref[...]
    mu = g[:, :D_Z]
    logstd = g[:, D_Z:]
    z = mu + eps_ref[...] * jnp.exp(logstd)
    mu_ref[...] = mu
    logstd_ref[...] = logstd
    z_ref[...] = z
    z2_ref[...] = z
    wnb = wnb_ref[...] * mask_ref[...]
    wzi = wzi_ref[...] * mask_ref[...]
    nb_logits = jnp.dot(z, wnb, preferred_element_type=jnp.float32)
    m = jnp.max(nb_logits, axis=1, keepdims=True)
    ex = jnp.exp(nb_logits - m)
    nb_ref[...] = lib_ref[...] * ex / jnp.sum(ex, axis=1, keepdims=True)
    zi_ref[...] = jnp.dot(z, wzi, preferred_element_type=jnp.float32)


def _dec_call(s2, t2, dinv, eps, lib, W_nb, W_zi, mask):
    grid = N // BR
    return pl.pallas_call(
        _dec_tc,
        grid=(grid,),
        in_specs=[
            pl.BlockSpec((NC, BR, EB), lambda i: (0, i, 0)),
            pl.BlockSpec((BR, D_H), lambda i: (i, 0)),
            pl.BlockSpec((BR, 1), lambda i: (i, 0)),
            pl.BlockSpec((BR, D_Z), lambda i: (i, 0)),
            pl.BlockSpec((BR, 1), lambda i: (i, 0)),
            pl.BlockSpec((D_Z, D_IN), lambda i: (0, 0)),
            pl.BlockSpec((D_Z, D_IN), lambda i: (0, 0)),
            pl.BlockSpec((D_Z, D_IN), lambda i: (0, 0)),
        ],
        out_specs=[
            pl.BlockSpec((BR, D_Z), lambda i: (i, 0)),
            pl.BlockSpec((BR, D_Z), lambda i: (i, 0)),
            pl.BlockSpec((BR, D_Z), lambda i: (i, 0)),
            pl.BlockSpec((BR, D_Z), lambda i: (i, 0)),
            pl.BlockSpec((BR, D_IN), lambda i: (i, 0)),
            pl.BlockSpec((BR, D_IN), lambda i: (i, 0)),
        ],
        out_shape=[
            jax.ShapeDtypeStruct((N, D_Z), jnp.float32),
            jax.ShapeDtypeStruct((N, D_Z), jnp.float32),
            jax.ShapeDtypeStruct((N, D_Z), jnp.float32),
            jax.ShapeDtypeStruct((N, D_Z), jnp.float32),
            jax.ShapeDtypeStruct((N, D_IN), jnp.float32),
            jax.ShapeDtypeStruct((N, D_IN), jnp.float32),
        ],
    )(s2, t2, dinv, eps, lib, W_nb, W_zi, mask)


def _adj_tc(zr_ref, zc_ref, adj_ref):
    adj_ref[...] = lax.dot_general(
        zr_ref[...], zc_ref[...], (((1,), (1,)), ((), ())),
        preferred_element_type=jnp.float32)


def _adj_call(z, z2):
    BRR = 200
    return pl.pallas_call(
        _adj_tc,
        grid=(N // BRR,),
        in_specs=[
            pl.BlockSpec((BRR, D_Z), lambda i: (i, 0)),
            pl.BlockSpec((N, D_Z), lambda i: (0, 0)),
        ],
        out_specs=pl.BlockSpec((BRR, N), lambda i: (i, 0)),
        out_shape=jax.ShapeDtypeStruct((N, N), jnp.float32),
    )(z, z2)


def kernel(x, edge_index, W1, b1, W_mu, b_mu, W_logstd, b_logstd,
           W_nb, W_zi, mask, eps):
    eib = edge_index.reshape(2, NBLK, EB)

    deg = _deg_call(eib)
    dinv = lax.rsqrt(deg[0, :N] + deg[1, :N] + 1.0).reshape(N, 1)
    t1, lib = _enc1_call(dinv, x, W1, b1.reshape(1, D_H))
    t1v = t1.reshape(2 * N, DQ)
    s1a = _prop_call(eib, t1v, DQ, 2, 0)
    s1b = _prop_call(eib, t1v, DQ, 2, 1)
    Wc = jnp.concatenate([W_mu, W_logstd], axis=1)
    bc = jnp.concatenate([b_mu, b_logstd]).reshape(1, 2 * D_Z)
    t2 = _enc2_call(s1a, s1b, t1, dinv, Wc, bc)
    t2v = t2.reshape(2 * N, 2 * D_Z)
    s2 = _prop_call(eib, t2v, 2 * D_Z, 2, 0)
    mu, logstd, z, z2, nb_means, zi = _dec_call(
        s2, t2, dinv, eps, lib, W_nb, W_zi, mask)
    adj = _adj_call(z, z2)
    return (adj, nb_means, zi, mu, logstd)
